# baseline (speedup 1.0000x reference)
import functools

import jax
import jax.numpy as jnp
from jax import lax
from jax.experimental import pallas as pl
from jax.experimental.pallas import tpu as pltpu
from jax.experimental.pallas import tpu_sc as plsc

B = 4096
L = 50
HIDDEN = 64
OUT = 128
VOCAB = 1000000
HALF = VOCAB // 2

NC = 2
NS = 16
NW = NC * NS
BPW = B // NW
ROWS_PER_CHUNK = 2
IDX_PER_CHUNK = ROWS_PER_CHUNK * L
NCHUNK = BPW // ROWS_PER_CHUNK
NROW2 = B // ROWS_PER_CHUNK
VPR = HIDDEN // 16
WCH = 16 * IDX_PER_CHUNK


def _accumulate(rows_ref, w_ref, pooled_ref, out_row):
    for r in range(ROWS_PER_CHUNK):
        base = r * L
        accs = [None] * VPR
        for l in range(L):
            j = base + l
            wv = w_ref[pl.ds(16 * j, 16)]
            for v in range(VPR):
                val = wv * rows_ref[j, pl.ds(16 * v, 16)]
                accs[v] = val if accs[v] is None else accs[v] + val
        for v in range(VPR):
            pooled_ref[out_row + r, pl.ds(16 * v, 16)] = accs[v]


def _pool_body(x2_hbm, w_hbm, table_hbm, out_hbm, idx_v, rows0, rows1,
               w0, w1, pooled_v, sem0, sem1):
    wid = lax.axis_index("s") * NC + lax.axis_index("c")
    base_irow = wid * NCHUNK

    pltpu.sync_copy(x2_hbm.at[pl.ds(base_irow, NCHUNK)], idx_v)

    def start(c, rows, wv, sem):
        pltpu.async_copy(table_hbm.at[idx_v.at[c]], rows, sem)
        pltpu.async_copy(w_hbm.at[pl.ds((base_irow + c) * WCH, WCH)], wv, sem)

    def wait(c, rows, wv, sem):
        pltpu.make_async_copy(table_hbm.at[idx_v.at[c]], rows, sem).wait()
        pltpu.make_async_copy(w_hbm.at[pl.ds(0, WCH)], wv, sem).wait()

    start(0, rows0, w0, sem0)
    start(1, rows1, w1, sem1)

    def body(i, carry):
        wait(2 * i, rows0, w0, sem0)
        _accumulate(rows0, w0, pooled_v, 4 * i)

        @pl.when(i < NCHUNK // 2 - 1)
        def _():
            start(2 * i + 2, rows0, w0, sem0)

        wait(2 * i + 1, rows1, w1, sem1)
        _accumulate(rows1, w1, pooled_v, 4 * i + 2)

        @pl.when(i < NCHUNK // 2 - 1)
        def _():
            start(2 * i + 3, rows1, w1, sem1)

        return carry

    lax.fori_loop(0, NCHUNK // 2, body, 0)

    pltpu.sync_copy(pooled_v, out_hbm.at[pl.ds(wid * BPW, BPW)])


_pool = functools.partial(
    pl.kernel,
    out_type=jax.ShapeDtypeStruct((B, HIDDEN), jnp.float32),
    mesh=plsc.VectorSubcoreMesh(core_axis_name="c", subcore_axis_name="s"),
    scratch_types=[
        pltpu.VMEM((NCHUNK, IDX_PER_CHUNK), jnp.int32),
        pltpu.VMEM((IDX_PER_CHUNK, HIDDEN), jnp.float32),
        pltpu.VMEM((IDX_PER_CHUNK, HIDDEN), jnp.float32),
        pltpu.VMEM((WCH,), jnp.float32),
        pltpu.VMEM((WCH,), jnp.float32),
        pltpu.VMEM((BPW, HIDDEN), jnp.float32),
        pltpu.SemaphoreType.DMA,
        pltpu.SemaphoreType.DMA,
    ],
    compiler_params=pltpu.CompilerParams(use_tc_tiling_on_sc=False),
)(_pool_body)


def _mm_body(ha_ref, hb_ref, w_ref, b_ref, o_ref):
    o_ref[...] = lax.dot_general(
        ha_ref[...] + hb_ref[...], w_ref[...],
        dimension_numbers=(((1,), (1,)), ((), ())),
        preferred_element_type=jnp.float32,
    ) + b_ref[...]


def _linear(ha, hb, w, bias):
    blk = 512
    return pl.pallas_call(
        _mm_body,
        grid=(B // blk,),
        in_specs=[
            pl.BlockSpec((blk, HIDDEN), lambda i: (i, 0)),
            pl.BlockSpec((blk, HIDDEN), lambda i: (i, 0)),
            pl.BlockSpec((OUT, HIDDEN), lambda i: (0, 0)),
            pl.BlockSpec((1, OUT), lambda i: (0, 0)),
        ],
        out_specs=pl.BlockSpec((blk, OUT), lambda i: (i, 0)),
        out_shape=jax.ShapeDtypeStruct((B, OUT), jnp.float32),
    )(ha, hb, w, bias)


def _weights(mask):
    return jnp.broadcast_to(
        mask.astype(jnp.float32).reshape(-1, 1),
        (NROW2 * IDX_PER_CHUNK, 16)).reshape(-1)


def kernel(x, emb_table, W, b):
    x2 = x.astype(jnp.int32).reshape(NROW2, IDX_PER_CHUNK)
    in_a = x2 < HALF
    xa = jnp.where(in_a, x2, 0)
    xb = jnp.where(in_a, 0, x2 - HALF)
    pooled_a = _pool(xa, _weights(in_a), emb_table[:HALF])
    pooled_b = _pool(xb, _weights(~in_a), emb_table[HALF:])
    return _linear(pooled_a, pooled_b, W, b.reshape(1, OUT))


# speedup vs baseline: 7.6164x; 7.6164x over previous
import functools

import jax
import jax.numpy as jnp
from jax import lax
from jax.experimental import pallas as pl
from jax.experimental.pallas import tpu as pltpu
from jax.experimental.pallas import tpu_sc as plsc

B = 4096
L = 50
HIDDEN = 64
OUT = 128

NC = 2
NS = 16
NW = NC * NS
BPW = B // NW
ROWS_PER_CHUNK = 2
IDX_PER_CHUNK = ROWS_PER_CHUNK * L
NCHUNK = BPW // ROWS_PER_CHUNK
VPR = HIDDEN // 16


def _accumulate(rows_ref, pooled_ref, out_row):
    for r in range(ROWS_PER_CHUNK):
        base = r * L
        accs = [rows_ref[base, pl.ds(16 * v, 16)] for v in range(VPR)]
        for l in range(1, L):
            for v in range(VPR):
                accs[v] = accs[v] + rows_ref[base + l, pl.ds(16 * v, 16)]
        for v in range(VPR):
            pooled_ref[out_row + r, pl.ds(16 * v, 16)] = accs[v]


def _pool_body(x2_hbm, table_hbm, out_hbm, idx_v, rows0, rows1, pooled_v,
               sem0, sem1):
    wid = lax.axis_index("s") * NC + lax.axis_index("c")
    base_irow = wid * NCHUNK

    pltpu.sync_copy(x2_hbm.at[pl.ds(base_irow, NCHUNK)], idx_v)

    pltpu.async_copy(table_hbm.at[idx_v.at[0]], rows0, sem0)
    pltpu.async_copy(table_hbm.at[idx_v.at[1]], rows1, sem1)

    def body(i, carry):
        pltpu.make_async_copy(table_hbm.at[idx_v.at[2 * i]], rows0, sem0).wait()
        _accumulate(rows0, pooled_v, 4 * i)

        @pl.when(i < NCHUNK // 2 - 1)
        def _():
            pltpu.async_copy(table_hbm.at[idx_v.at[2 * i + 2]], rows0, sem0)

        pltpu.make_async_copy(table_hbm.at[idx_v.at[2 * i + 1]], rows1,
                              sem1).wait()
        _accumulate(rows1, pooled_v, 4 * i + 2)

        @pl.when(i < NCHUNK // 2 - 1)
        def _():
            pltpu.async_copy(table_hbm.at[idx_v.at[2 * i + 3]], rows1, sem1)

        return carry

    lax.fori_loop(0, NCHUNK // 2, body, 0)

    pltpu.sync_copy(pooled_v, out_hbm.at[pl.ds(wid * BPW, BPW)])


_pool = functools.partial(
    pl.kernel,
    out_type=jax.ShapeDtypeStruct((B, HIDDEN), jnp.float32),
    mesh=plsc.VectorSubcoreMesh(core_axis_name="c", subcore_axis_name="s"),
    scratch_types=[
        pltpu.VMEM((NCHUNK, IDX_PER_CHUNK), jnp.int32),
        pltpu.VMEM((IDX_PER_CHUNK, HIDDEN), jnp.float32),
        pltpu.VMEM((IDX_PER_CHUNK, HIDDEN), jnp.float32),
        pltpu.VMEM((BPW, HIDDEN), jnp.float32),
        pltpu.SemaphoreType.DMA,
        pltpu.SemaphoreType.DMA,
    ],
    compiler_params=pltpu.CompilerParams(use_tc_tiling_on_sc=False),
)(_pool_body)


def _mm_body(h_ref, w_ref, b_ref, o_ref):
    o_ref[...] = lax.dot_general(
        h_ref[...], w_ref[...],
        dimension_numbers=(((1,), (1,)), ((), ())),
        preferred_element_type=jnp.float32,
    ) + b_ref[...]


def _linear(h, w, bias):
    blk = 512
    return pl.pallas_call(
        _mm_body,
        grid=(B // blk,),
        in_specs=[
            pl.BlockSpec((blk, HIDDEN), lambda i: (i, 0)),
            pl.BlockSpec((OUT, HIDDEN), lambda i: (0, 0)),
            pl.BlockSpec((1, OUT), lambda i: (0, 0)),
        ],
        out_specs=pl.BlockSpec((blk, OUT), lambda i: (i, 0)),
        out_shape=jax.ShapeDtypeStruct((B, OUT), jnp.float32),
    )(h, w, bias)


def kernel(x, emb_table, W, b):
    x2 = x.astype(jnp.int32).reshape(B // ROWS_PER_CHUNK, IDX_PER_CHUNK)
    pooled = _pool(x2, emb_table)
    return _linear(pooled, W, b.reshape(1, OUT))
